# baseline (speedup 1.0000x reference)
import functools

import jax
import jax.numpy as jnp
from jax import lax
from jax.experimental import pallas as pl
from jax.experimental.pallas import tpu as pltpu
from jax.experimental.pallas import tpu_sc as plsc

_K = 128
_N = 32768
_L = 16
_STEPS = _N // _L
_HI0 = 0x3F800001
_FULL_ITERS = 10
_TAIL_ITERS = 21


def _process_row(row, row_v, cand_v, tie_v, k_s, scores_hbm, out_hbm):
    pltpu.sync_copy(scores_hbm.at[row], row_v)
    iota = lax.iota(jnp.int32, _L)
    zero_i = jnp.zeros((_L,), jnp.int32)

    mid0 = _HI0 >> 1

    def p1(i, carry):
        sacc, cacc = carry
        x = row_v[pl.ds(i * _L, _L)]
        xc = jnp.clip(x, 0.0, 1.0)
        row_v[pl.ds(i * _L, _L)] = xc
        xb = plsc.bitcast(xc, jnp.int32)
        return sacc + xc, cacc + jnp.where(xb >= mid0, 1, 0)

    sacc, cacc = lax.fori_loop(0, _STEPS, p1, (jnp.zeros((_L,), jnp.float32), zero_i))
    s_row = jnp.sum(sacc)
    cnt = jnp.sum(cacc)
    ge = cnt >= _K
    lo = jnp.where(ge, mid0, 0)
    hi = jnp.where(ge, _HI0, mid0)
    cnt_lo = jnp.where(ge, cnt, _N)
    cnt_hi = jnp.where(ge, 0, cnt)

    def fullit(_, carry):
        lo, hi, cnt_lo, cnt_hi = carry
        mid = (lo + hi) >> 1

        def cbody(i, cacc):
            xb = plsc.bitcast(row_v[pl.ds(i * _L, _L)], jnp.int32)
            return cacc + jnp.where(xb >= mid, 1, 0)

        cnt = jnp.sum(lax.fori_loop(0, _STEPS, cbody, zero_i))
        ge = cnt >= _K
        return (jnp.where(ge, mid, lo), jnp.where(ge, hi, mid),
                jnp.where(ge, cnt, cnt_lo), jnp.where(ge, cnt_hi, cnt))

    lo, hi, cnt_lo, cnt_hi = lax.fori_loop(
        0, _FULL_ITERS - 1, fullit, (lo, hi, cnt_lo, cnt_hi))

    mid = (lo + hi) >> 1
    base = cnt_hi

    def compact(i, carry):
        wvec, cacc = carry
        x = row_v[pl.ds(i * _L, _L)]
        xb = plsc.bitcast(x, jnp.int32)
        m = (xb >= lo) & (xb < hi)
        cacc = cacc + jnp.where(m & (xb >= mid), 1, 0)
        csum = plsc.cumsum(jnp.where(m, 1, 0))
        plsc.store_scatter(cand_v, [wvec + csum - 1], x, mask=m)
        return wvec + plsc.all_reduce_population_count(m), cacc

    wvec, cacc = lax.fori_loop(0, _STEPS, compact, (zero_i, zero_i))
    cs = jnp.sum(wvec) >> 4
    cnt = jnp.sum(cacc) + base
    ge = cnt >= _K
    lo = jnp.where(ge, mid, lo)
    hi_new = jnp.where(ge, hi, mid)
    cnt_lo = jnp.where(ge, cnt, cnt_lo)
    cnt_hi = jnp.where(ge, cnt_hi, cnt)
    hi = hi_new

    nsteps = (cs + _L - 1) // _L

    def tailit(_, carry):
        lo, hi, cnt_lo, cnt_hi = carry
        mid = (lo + hi) >> 1

        def cbody(i, cacc):
            xb = plsc.bitcast(cand_v[pl.ds(i * _L, _L)], jnp.int32)
            valid = (i * _L + iota) < cs
            return cacc + jnp.where(valid & (xb >= mid), 1, 0)

        cnt = jnp.sum(lax.fori_loop(0, nsteps, cbody, zero_i)) + base
        ge = cnt >= _K
        return (jnp.where(ge, mid, lo), jnp.where(ge, hi, mid),
                jnp.where(ge, cnt, cnt_lo), jnp.where(ge, cnt_hi, cnt))

    lo, hi, cnt_lo, cnt_hi = lax.fori_loop(
        0, _TAIL_ITERS, tailit, (lo, hi, cnt_lo, cnt_hi))

    t = lo
    n_gt = cnt_hi
    need = _K - n_gt
    cond = s_row > k_s

    @pl.when(cond & (cnt_lo == _K))
    def _():
        def obody(i, c):
            x = row_v[pl.ds(i * _L, _L)]
            xb = plsc.bitcast(x, jnp.int32)
            row_v[pl.ds(i * _L, _L)] = jnp.where(xb >= t, x, 0.0)
            return c

        lax.fori_loop(0, _STEPS, obody, 0)

    @pl.when(cond & (cnt_lo != _K))
    def _():
        def obody(i, wvec):
            x = row_v[pl.ds(i * _L, _L)]
            xb = plsc.bitcast(x, jnp.int32)
            meq = xb == t
            csum = plsc.cumsum(jnp.where(meq, 1, 0))
            plsc.store_scatter(tie_v, [wvec + csum - 1], i * _L + iota, mask=meq)
            row_v[pl.ds(i * _L, _L)] = jnp.where(xb > t, x, 0.0)
            return wvec + plsc.all_reduce_population_count(meq)

        lax.fori_loop(0, _STEPS, obody, zero_i)
        tvals = plsc.bitcast(jnp.broadcast_to(t, (_L,)), jnp.float32)

        def rbody(i, c):
            tix = tie_v[pl.ds(i * _L, _L)]
            valid = (i * _L + iota) < need
            plsc.store_scatter(row_v, [tix], tvals, mask=valid)
            return c

        lax.fori_loop(0, (need + _L - 1) // _L, rbody, 0)

    pltpu.sync_copy(row_v, out_hbm.at[row])


def _sc_body(scores_hbm, kvec_hbm, out_hbm, row_v, cand_v, tie_v, kv_v):
    wid = lax.axis_index("s") * 2 + lax.axis_index("c")
    pltpu.sync_copy(kvec_hbm, kv_v)
    k_s = jnp.sum(kv_v[...]) * 0.0625
    for r in range(2):
        _process_row(wid * 2 + r, row_v, cand_v, tie_v, k_s,
                     scores_hbm, out_hbm)


def kernel(scores, k):
    kvec = jnp.broadcast_to(jnp.asarray(k, jnp.float32), (_L,))
    mesh = plsc.VectorSubcoreMesh(core_axis_name="c", subcore_axis_name="s")
    fn = functools.partial(
        pl.kernel,
        mesh=mesh,
        out_type=jax.ShapeDtypeStruct(scores.shape, scores.dtype),
        scratch_types=[
            pltpu.VMEM((_N,), jnp.float32),
            pltpu.VMEM((_N,), jnp.float32),
            pltpu.VMEM((_N,), jnp.int32),
            pltpu.VMEM((_L,), jnp.float32),
        ],
        compiler_params=pltpu.CompilerParams(needs_layout_passes=False),
    )(_sc_body)
    return fn(scores, kvec)


# speedup vs baseline: 1.9409x; 1.9409x over previous
import functools

import jax
import jax.numpy as jnp
from jax import lax
from jax.experimental import pallas as pl
from jax.experimental.pallas import tpu as pltpu
from jax.experimental.pallas import tpu_sc as plsc

_K = 128
_N = 32768
_L = 16
_HI0 = 0x3F800001
_NB = 128
_U = 8


def _process_row(row, row_v, cand_v, tie_v, hist_v, k_s, scores_hbm, out_hbm):
    pltpu.sync_copy(scores_hbm.at[row], row_v)
    iota = lax.iota(jnp.int32, _L)
    ones_i = jnp.ones((_L,), jnp.int32)
    zero_i = jnp.zeros((_L,), jnp.int32)

    def zbody(i, c):
        hist_v[pl.ds(i * _L, _L)] = zero_i
        return c

    lax.fori_loop(0, _NB, zbody, 0)

    def p1(i, saccs):
        out = []
        for u in range(_U):
            x = row_v[pl.ds((i * _U + u) * _L, _L)]
            xc = jnp.clip(x, 0.0, 1.0)
            out.append(saccs[u] + xc)
            e = plsc.bitcast(1.0 - xc, jnp.int32) >> 23
            plsc.addupdate_scatter(hist_v, [(e << 4) + iota], ones_i)
        return tuple(out)

    saccs = lax.fori_loop(
        0, _N // (_L * _U), p1,
        tuple(jnp.zeros((_L,), jnp.float32) for _ in range(_U)))
    s_row = jnp.sum(sum(saccs))

    carry = jnp.int32(0)
    bstar = jnp.int32(0)
    base = jnp.int32(0)
    for g in range(_NB // _L):
        tot = zero_i
        bidx = (g * _L + iota) << 4
        for p in range(_L):
            tot = tot + plsc.load_gather(hist_v, [bidx + p])
        cum = plsc.cumsum(tot) + carry
        carry = cum[_L - 1]
        below = cum < _K
        bstar = bstar + plsc.all_reduce_population_count(below)[0]
        base = jnp.maximum(base, jnp.max(jnp.where(below, cum, 0)))

    def p2(i, wvec):
        for u in range(_U):
            x = row_v[pl.ds((i * _U + u) * _L, _L)]
            xc = jnp.clip(x, 0.0, 1.0)
            e = plsc.bitcast(1.0 - xc, jnp.int32) >> 23
            m = e == bstar
            csum = plsc.cumsum(jnp.where(m, 1, 0))
            plsc.store_scatter(cand_v, [wvec + csum - 1], xc, mask=m)
            wvec = wvec + plsc.all_reduce_population_count(m)
        return wvec

    wvec = lax.fori_loop(0, _N // (_L * _U), p2, zero_i)
    cs = wvec[0]

    nsteps = (cs + _L - 1) // _L

    def bit(_, carryv):
        lo, hi, cnt_lo, cnt_hi = carryv
        mid = (lo + hi) >> 1

        def cbody(i, cacc):
            xb = plsc.bitcast(cand_v[pl.ds(i * _L, _L)], jnp.int32)
            valid = (i * _L + iota) < cs
            return cacc + jnp.where(valid & (xb >= mid), 1, 0)

        cnt = jnp.sum(lax.fori_loop(0, nsteps, cbody, zero_i)) + base
        ge = cnt >= _K
        return (jnp.where(ge, mid, lo), jnp.where(ge, hi, mid),
                jnp.where(ge, cnt, cnt_lo), jnp.where(ge, cnt_hi, cnt))

    lo, hi, cnt_lo, cnt_hi = lax.fori_loop(
        0, 31, bit, (jnp.int32(0), jnp.int32(_HI0),
                     base + cs, jnp.int32(0)))

    t = lo
    n_gt = cnt_hi
    need = _K - n_gt
    cond = s_row > k_s

    @pl.when(cond & (cnt_lo == _K))
    def _():
        def obody(i, c):
            for u in range(_U):
                sl = pl.ds((i * _U + u) * _L, _L)
                xc = jnp.clip(row_v[sl], 0.0, 1.0)
                xb = plsc.bitcast(xc, jnp.int32)
                row_v[sl] = jnp.where(xb >= t, xc, 0.0)
            return c

        lax.fori_loop(0, _N // (_L * _U), obody, 0)

    @pl.when(cond & (cnt_lo != _K))
    def _():
        def obody(i, wv):
            x = row_v[pl.ds(i * _L, _L)]
            xc = jnp.clip(x, 0.0, 1.0)
            xb = plsc.bitcast(xc, jnp.int32)
            meq = xb == t
            csum = plsc.cumsum(jnp.where(meq, 1, 0))
            plsc.store_scatter(tie_v, [wv + csum - 1], i * _L + iota, mask=meq)
            row_v[pl.ds(i * _L, _L)] = jnp.where(xb > t, xc, 0.0)
            return wv + plsc.all_reduce_population_count(meq)

        lax.fori_loop(0, _N // _L, obody, zero_i)
        tvals = plsc.bitcast(jnp.broadcast_to(t, (_L,)), jnp.float32)

        def rbody(i, c):
            tix = tie_v[pl.ds(i * _L, _L)]
            valid = (i * _L + iota) < need
            plsc.store_scatter(row_v, [tix], tvals, mask=valid)
            return c

        lax.fori_loop(0, (need + _L - 1) // _L, rbody, 0)

    @pl.when(jnp.logical_not(cond))
    def _():
        def obody(i, c):
            for u in range(_U):
                sl = pl.ds((i * _U + u) * _L, _L)
                row_v[sl] = jnp.clip(row_v[sl], 0.0, 1.0)
            return c

        lax.fori_loop(0, _N // (_L * _U), obody, 0)

    pltpu.sync_copy(row_v, out_hbm.at[row])


def _sc_body(scores_hbm, kvec_hbm, out_hbm, row_v, cand_v, tie_v, hist_v, kv_v):
    wid = lax.axis_index("s") * 2 + lax.axis_index("c")
    pltpu.sync_copy(kvec_hbm, kv_v)
    k_s = jnp.sum(kv_v[...]) * 0.0625
    for r in range(2):
        _process_row(wid * 2 + r, row_v, cand_v, tie_v, hist_v, k_s,
                     scores_hbm, out_hbm)


def kernel(scores, k):
    kvec = jnp.broadcast_to(jnp.asarray(k, jnp.float32), (_L,))
    mesh = plsc.VectorSubcoreMesh(core_axis_name="c", subcore_axis_name="s")
    fn = functools.partial(
        pl.kernel,
        mesh=mesh,
        out_type=jax.ShapeDtypeStruct(scores.shape, scores.dtype),
        scratch_types=[
            pltpu.VMEM((_N,), jnp.float32),
            pltpu.VMEM((_N,), jnp.float32),
            pltpu.VMEM((_N,), jnp.int32),
            pltpu.VMEM((_NB * _L,), jnp.int32),
            pltpu.VMEM((_L,), jnp.float32),
        ],
        compiler_params=pltpu.CompilerParams(needs_layout_passes=False),
    )(_sc_body)
    return fn(scores, kvec)


# speedup vs baseline: 3.0940x; 1.5941x over previous
import functools

import jax
import jax.numpy as jnp
from jax import lax
from jax.experimental import pallas as pl
from jax.experimental.pallas import tpu as pltpu
from jax.experimental.pallas import tpu_sc as plsc

_K = 128
_N = 32768
_L = 16
_HI0 = 0x3F800001
_NB = 128
_SEG = _N // _L
_U = 4


def _process_row(row, row_v, cand_v, tie_v, hist_v, k_s, scores_hbm, out_hbm):
    pltpu.sync_copy(scores_hbm.at[row], row_v)
    iota = lax.iota(jnp.int32, _L)
    ones_i = jnp.ones((_L,), jnp.int32)
    zero_i = jnp.zeros((_L,), jnp.int32)

    @plsc.parallel_loop(0, _NB, unroll=4)
    def _(i):
        hist_v[pl.ds(i * _L, _L)] = zero_i

    @plsc.parallel_loop(
        0, _N // (_L * _U), unroll=2,
        carry=tuple(jnp.zeros((_L,), jnp.float32) for _ in range(_U)))
    def saccs(i, saccs):
        out = []
        for u in range(_U):
            x = row_v[pl.ds((i * _U + u) * _L, _L)]
            xc = jnp.clip(x, 0.0, 1.0)
            out.append(saccs[u] + xc)
            e = plsc.bitcast(1.0 - xc, jnp.int32) >> 23
            plsc.addupdate_scatter(hist_v, [(e << 4) + iota], ones_i)
        return tuple(out)

    s_row = jnp.sum(sum(saccs))

    carry = jnp.int32(0)
    bstar = jnp.int32(0)
    base = jnp.int32(0)
    for g in range(_NB // _L):
        tot = zero_i
        bidx = (g * _L + iota) << 4
        for p in range(_L):
            tot = tot + plsc.load_gather(hist_v, [bidx + p])
        cum = plsc.cumsum(tot) + carry
        carry = cum[_L - 1]
        below = cum < _K
        bstar = bstar + plsc.all_reduce_population_count(below)[0]
        base = jnp.maximum(base, jnp.max(jnp.where(below, cum, 0)))

    seg_base = iota * _SEG

    @plsc.parallel_loop(0, _N // (_L * _U), unroll=2, carry=zero_i)
    def percnt(i, percnt):
        for u in range(_U):
            x = row_v[pl.ds((i * _U + u) * _L, _L)]
            xc = jnp.clip(x, 0.0, 1.0)
            e = plsc.bitcast(1.0 - xc, jnp.int32) >> 23
            m = e == bstar
            plsc.store_scatter(cand_v, [seg_base + percnt], xc, mask=m)
            percnt = percnt + jnp.where(m, 1, 0)
        return percnt

    cs_max = jnp.max(percnt)
    cs_tot = jnp.sum(percnt)

    def bit(_, carryv):
        lo, hi, cnt_lo, cnt_hi = carryv
        mid = (lo + hi) >> 1

        def cbody(i, cacc):
            xb = plsc.bitcast(plsc.load_gather(cand_v, [seg_base + i]),
                              jnp.int32)
            valid = i < percnt
            return cacc + jnp.where(valid & (xb >= mid), 1, 0)

        cnt = jnp.sum(lax.fori_loop(0, cs_max, cbody, zero_i)) + base
        ge = cnt >= _K
        return (jnp.where(ge, mid, lo), jnp.where(ge, hi, mid),
                jnp.where(ge, cnt, cnt_lo), jnp.where(ge, cnt_hi, cnt))

    lo, hi, cnt_lo, cnt_hi = lax.fori_loop(
        0, 31, bit, (jnp.int32(0), jnp.int32(_HI0),
                     base + cs_tot, jnp.int32(0)))

    t = lo
    n_gt = cnt_hi
    need = _K - n_gt
    cond = s_row > k_s

    @pl.when(cond & (cnt_lo == _K))
    def _():
        @plsc.parallel_loop(0, _N // (_L * _U), unroll=2)
        def _(i):
            for u in range(_U):
                sl = pl.ds((i * _U + u) * _L, _L)
                xc = jnp.clip(row_v[sl], 0.0, 1.0)
                xb = plsc.bitcast(xc, jnp.int32)
                row_v[sl] = jnp.where(xb >= t, xc, 0.0)

    @pl.when(cond & (cnt_lo != _K))
    def _():
        def obody(i, wv):
            x = row_v[pl.ds(i * _L, _L)]
            xc = jnp.clip(x, 0.0, 1.0)
            xb = plsc.bitcast(xc, jnp.int32)
            meq = xb == t
            csum = plsc.cumsum(jnp.where(meq, 1, 0))
            plsc.store_scatter(tie_v, [wv + csum - 1], i * _L + iota, mask=meq)
            row_v[pl.ds(i * _L, _L)] = jnp.where(xb > t, xc, 0.0)
            return wv + plsc.all_reduce_population_count(meq)

        lax.fori_loop(0, _N // _L, obody, zero_i)
        tvals = plsc.bitcast(jnp.broadcast_to(t, (_L,)), jnp.float32)

        def rbody(i, c):
            tix = tie_v[pl.ds(i * _L, _L)]
            valid = (i * _L + iota) < need
            plsc.store_scatter(row_v, [tix], tvals, mask=valid)
            return c

        lax.fori_loop(0, (need + _L - 1) // _L, rbody, 0)

    @pl.when(jnp.logical_not(cond))
    def _():
        @plsc.parallel_loop(0, _N // (_L * _U), unroll=2)
        def _(i):
            for u in range(_U):
                sl = pl.ds((i * _U + u) * _L, _L)
                row_v[sl] = jnp.clip(row_v[sl], 0.0, 1.0)

    pltpu.sync_copy(row_v, out_hbm.at[row])


def _sc_body(scores_hbm, kvec_hbm, out_hbm, row_v, cand_v, tie_v, hist_v, kv_v):
    wid = lax.axis_index("s") * 2 + lax.axis_index("c")
    pltpu.sync_copy(kvec_hbm, kv_v)
    k_s = jnp.sum(kv_v[...]) * 0.0625
    for r in range(2):
        _process_row(wid * 2 + r, row_v, cand_v, tie_v, hist_v, k_s,
                     scores_hbm, out_hbm)


def kernel(scores, k):
    kvec = jnp.broadcast_to(jnp.asarray(k, jnp.float32), (_L,))
    mesh = plsc.VectorSubcoreMesh(core_axis_name="c", subcore_axis_name="s")
    fn = functools.partial(
        pl.kernel,
        mesh=mesh,
        out_type=jax.ShapeDtypeStruct(scores.shape, scores.dtype),
        scratch_types=[
            pltpu.VMEM((_N,), jnp.float32),
            pltpu.VMEM((_N,), jnp.float32),
            pltpu.VMEM((_N,), jnp.int32),
            pltpu.VMEM((_NB * _L,), jnp.int32),
            pltpu.VMEM((_L,), jnp.float32),
        ],
        compiler_params=pltpu.CompilerParams(needs_layout_passes=False),
    )(_sc_body)
    return fn(scores, kvec)


# speedup vs baseline: 3.3044x; 1.0680x over previous
import functools

import jax
import jax.numpy as jnp
from jax import lax
from jax.experimental import pallas as pl
from jax.experimental.pallas import tpu as pltpu
from jax.experimental.pallas import tpu_sc as plsc

_K = 128
_N = 32768
_L = 16
_HI0 = 0x3F800001
_NB = 128
_SEG = _N // _L
_U = 4


def _process_row(row, row_v, cand_v, tie_v, hist_v, k_s, scores_hbm, out_hbm):
    pltpu.sync_copy(scores_hbm.at[row], row_v)
    iota = lax.iota(jnp.int32, _L)
    ones_i = jnp.ones((_L,), jnp.int32)
    zero_i = jnp.zeros((_L,), jnp.int32)

    @plsc.parallel_loop(0, _NB, unroll=4)
    def _(i):
        hist_v[pl.ds(i * _L, _L)] = zero_i

    @plsc.parallel_loop(
        0, _N // (_L * _U), unroll=2,
        carry=tuple(jnp.zeros((_L,), jnp.float32) for _ in range(_U)))
    def saccs(i, saccs):
        out = []
        for u in range(_U):
            x = row_v[pl.ds((i * _U + u) * _L, _L)]
            xc = jnp.clip(x, 0.0, 1.0)
            out.append(saccs[u] + xc)
            e = plsc.bitcast(1.0 - xc, jnp.int32) >> 23
            plsc.addupdate_scatter(hist_v, [(e << 4) + iota], ones_i)
        return tuple(out)

    s_row = jnp.sum(sum(saccs))

    carry = jnp.int32(0)
    bstar = jnp.int32(0)
    base = jnp.int32(0)
    for g in range(_NB // _L):
        tot = zero_i
        bidx = (g * _L + iota) << 4
        for p in range(_L):
            tot = tot + plsc.load_gather(hist_v, [bidx + p])
        cum = plsc.cumsum(tot) + carry
        carry = cum[_L - 1]
        below = cum < _K
        bstar = bstar + plsc.all_reduce_population_count(below)[0]
        base = jnp.maximum(base, jnp.max(jnp.where(below, cum, 0)))

    seg_base = iota * _SEG

    @plsc.parallel_loop(
        0, _N // (_L * _U), unroll=2,
        carry=(zero_i, jnp.full((_L,), _HI0, jnp.int32), zero_i))
    def p2carry(i, carryv):
        percnt, cmin, cmax = carryv
        for u in range(_U):
            x = row_v[pl.ds((i * _U + u) * _L, _L)]
            xc = jnp.clip(x, 0.0, 1.0)
            xb = plsc.bitcast(xc, jnp.int32)
            e = plsc.bitcast(1.0 - xc, jnp.int32) >> 23
            m = e == bstar
            plsc.store_scatter(cand_v, [seg_base + percnt], xc, mask=m)
            percnt = percnt + jnp.where(m, 1, 0)
            cmin = jnp.where(m, jnp.minimum(cmin, xb), cmin)
            cmax = jnp.where(m, jnp.maximum(cmax, xb), cmax)
        return percnt, cmin, cmax

    percnt, cmin, cmax = p2carry
    cs_max = jnp.max(percnt)
    cs_tot = jnp.sum(percnt)

    def bit(carryv):
        lo, hi, cnt_lo, cnt_hi = carryv
        mid = (lo + hi) >> 1

        def cbody(i, cacc):
            xb = plsc.bitcast(plsc.load_gather(cand_v, [seg_base + i]),
                              jnp.int32)
            valid = i < percnt
            return cacc + jnp.where(valid & (xb >= mid), 1, 0)

        cnt = jnp.sum(lax.fori_loop(0, cs_max, cbody, zero_i)) + base
        ge = cnt >= _K
        return (jnp.where(ge, mid, lo), jnp.where(ge, hi, mid),
                jnp.where(ge, cnt, cnt_lo), jnp.where(ge, cnt_hi, cnt))

    lo, hi, cnt_lo, cnt_hi = lax.while_loop(
        lambda c: c[1] - c[0] > 1, bit,
        (jnp.min(cmin), jnp.max(cmax) + 1, base + cs_tot, base))

    t = lo
    n_gt = cnt_hi
    need = _K - n_gt
    cond = s_row > k_s

    @pl.when(cond & (cnt_lo == _K))
    def _():
        @plsc.parallel_loop(0, _N // (_L * _U), unroll=2)
        def _(i):
            for u in range(_U):
                sl = pl.ds((i * _U + u) * _L, _L)
                xc = jnp.clip(row_v[sl], 0.0, 1.0)
                xb = plsc.bitcast(xc, jnp.int32)
                row_v[sl] = jnp.where(xb >= t, xc, 0.0)

    @pl.when(cond & (cnt_lo != _K))
    def _():
        def obody(i, wv):
            x = row_v[pl.ds(i * _L, _L)]
            xc = jnp.clip(x, 0.0, 1.0)
            xb = plsc.bitcast(xc, jnp.int32)
            meq = xb == t
            csum = plsc.cumsum(jnp.where(meq, 1, 0))
            plsc.store_scatter(tie_v, [wv + csum - 1], i * _L + iota, mask=meq)
            row_v[pl.ds(i * _L, _L)] = jnp.where(xb > t, xc, 0.0)
            return wv + plsc.all_reduce_population_count(meq)

        lax.fori_loop(0, _N // _L, obody, zero_i)
        tvals = plsc.bitcast(jnp.broadcast_to(t, (_L,)), jnp.float32)

        def rbody(i, c):
            tix = tie_v[pl.ds(i * _L, _L)]
            valid = (i * _L + iota) < need
            plsc.store_scatter(row_v, [tix], tvals, mask=valid)
            return c

        lax.fori_loop(0, (need + _L - 1) // _L, rbody, 0)

    @pl.when(jnp.logical_not(cond))
    def _():
        @plsc.parallel_loop(0, _N // (_L * _U), unroll=2)
        def _(i):
            for u in range(_U):
                sl = pl.ds((i * _U + u) * _L, _L)
                row_v[sl] = jnp.clip(row_v[sl], 0.0, 1.0)

    pltpu.sync_copy(row_v, out_hbm.at[row])


def _sc_body(scores_hbm, kvec_hbm, out_hbm, row_v, cand_v, tie_v, hist_v, kv_v):
    wid = lax.axis_index("s") * 2 + lax.axis_index("c")
    pltpu.sync_copy(kvec_hbm, kv_v)
    k_s = jnp.sum(kv_v[...]) * 0.0625
    for r in range(2):
        _process_row(wid * 2 + r, row_v, cand_v, tie_v, hist_v, k_s,
                     scores_hbm, out_hbm)


def kernel(scores, k):
    kvec = jnp.broadcast_to(jnp.asarray(k, jnp.float32), (_L,))
    mesh = plsc.VectorSubcoreMesh(core_axis_name="c", subcore_axis_name="s")
    fn = functools.partial(
        pl.kernel,
        mesh=mesh,
        out_type=jax.ShapeDtypeStruct(scores.shape, scores.dtype),
        scratch_types=[
            pltpu.VMEM((_N,), jnp.float32),
            pltpu.VMEM((_N,), jnp.float32),
            pltpu.VMEM((_N,), jnp.int32),
            pltpu.VMEM((_NB * _L,), jnp.int32),
            pltpu.VMEM((_L,), jnp.float32),
        ],
        compiler_params=pltpu.CompilerParams(needs_layout_passes=False),
    )(_sc_body)
    return fn(scores, kvec)


# speedup vs baseline: 4.0733x; 1.2327x over previous
import functools

import jax
import jax.numpy as jnp
from jax import lax
from jax.experimental import pallas as pl
from jax.experimental.pallas import tpu as pltpu
from jax.experimental.pallas import tpu_sc as plsc

_K = 128
_N = 32768
_L = 16
_HI0 = 0x3F800001
_NB = 128
_SEG2 = _N // (_L * 4)
_U = 4


def _process_row(row, row_v, cand_v, tie_v, hist_v, k_s, scores_hbm, out_hbm):
    pltpu.sync_copy(scores_hbm.at[row], row_v)
    iota = lax.iota(jnp.int32, _L)
    ones_i = jnp.ones((_L,), jnp.int32)
    zero_i = jnp.zeros((_L,), jnp.int32)

    @plsc.parallel_loop(0, _NB, unroll=4)
    def _(i):
        hist_v[pl.ds(i * _L, _L)] = zero_i

    @plsc.parallel_loop(
        0, _N // (_L * _U), unroll=2,
        carry=tuple(jnp.zeros((_L,), jnp.float32) for _ in range(_U)))
    def saccs(i, saccs):
        out = []
        for u in range(_U):
            x = row_v[pl.ds((i * _U + u) * _L, _L)]
            xc = jnp.clip(x, 0.0, 1.0)
            out.append(saccs[u] + xc)
            e = plsc.bitcast(1.0 - xc, jnp.int32) >> 23
            plsc.addupdate_scatter(hist_v, [(e << 4) + iota], ones_i)
        return tuple(out)

    s_row = jnp.sum(sum(saccs))

    carry = jnp.int32(0)
    bstar = jnp.int32(0)
    base = jnp.int32(0)
    for g in range(_NB // _L):
        tot = zero_i
        bidx = (g * _L + iota) << 4
        for p in range(_L):
            tot = tot + plsc.load_gather(hist_v, [bidx + p])
        cum = plsc.cumsum(tot) + carry
        carry = cum[_L - 1]
        below = cum < _K
        bstar = bstar + plsc.all_reduce_population_count(below)[0]
        base = jnp.maximum(base, jnp.max(jnp.where(below, cum, 0)))

    seg_bases = [(iota * _U + u) * _SEG2 for u in range(_U)]

    @plsc.parallel_loop(
        0, _N // (_L * _U), unroll=2,
        carry=tuple(zero_i for _ in range(_U)))
    def percnts(i, percnts):
        out = []
        for u in range(_U):
            x = row_v[pl.ds((i * _U + u) * _L, _L)]
            xc = jnp.clip(x, 0.0, 1.0)
            e = plsc.bitcast(1.0 - xc, jnp.int32) >> 23
            m = e == bstar
            plsc.store_scatter(cand_v, [seg_bases[u] + percnts[u]], xc, mask=m)
            out.append(percnts[u] + jnp.where(m, 1, 0))
        return tuple(out)

    cs_max = jnp.max(jnp.maximum(jnp.maximum(percnts[0], percnts[1]),
                                 jnp.maximum(percnts[2], percnts[3])))
    cs_tot = jnp.sum(percnts[0] + percnts[1] + percnts[2] + percnts[3])

    def bit(carryv):
        lo, hi, cnt_lo, cnt_hi = carryv
        mid = (lo + hi) >> 1

        def cbody(i, cacc):
            for u in range(_U):
                xb = plsc.bitcast(
                    plsc.load_gather(cand_v, [seg_bases[u] + i]), jnp.int32)
                valid = i < percnts[u]
                cacc = cacc + jnp.where(valid & (xb >= mid), 1, 0)
            return cacc

        cnt = jnp.sum(lax.fori_loop(0, cs_max, cbody, zero_i)) + base
        ge = cnt >= _K
        return (jnp.where(ge, mid, lo), jnp.where(ge, hi, mid),
                jnp.where(ge, cnt, cnt_lo), jnp.where(ge, cnt_hi, cnt))

    lo, hi, cnt_lo, cnt_hi = lax.while_loop(
        lambda c: c[1] - c[0] > 1, bit,
        (jnp.int32(0), jnp.int32(_HI0), base + cs_tot, jnp.int32(0)))

    t = lo
    n_gt = cnt_hi
    need = _K - n_gt
    cond = s_row > k_s

    @pl.when(cond & (cnt_lo == _K))
    def _():
        @plsc.parallel_loop(0, _N // (_L * _U), unroll=2)
        def _(i):
            for u in range(_U):
                sl = pl.ds((i * _U + u) * _L, _L)
                xc = jnp.clip(row_v[sl], 0.0, 1.0)
                xb = plsc.bitcast(xc, jnp.int32)
                row_v[sl] = jnp.where(xb >= t, xc, 0.0)

    @pl.when(cond & (cnt_lo != _K))
    def _():
        def obody(i, wv):
            x = row_v[pl.ds(i * _L, _L)]
            xc = jnp.clip(x, 0.0, 1.0)
            xb = plsc.bitcast(xc, jnp.int32)
            meq = xb == t
            csum = plsc.cumsum(jnp.where(meq, 1, 0))
            plsc.store_scatter(tie_v, [wv + csum - 1], i * _L + iota, mask=meq)
            row_v[pl.ds(i * _L, _L)] = jnp.where(xb > t, xc, 0.0)
            return wv + plsc.all_reduce_population_count(meq)

        lax.fori_loop(0, _N // _L, obody, zero_i)
        tvals = plsc.bitcast(jnp.broadcast_to(t, (_L,)), jnp.float32)

        def rbody(i, c):
            tix = tie_v[pl.ds(i * _L, _L)]
            valid = (i * _L + iota) < need
            plsc.store_scatter(row_v, [tix], tvals, mask=valid)
            return c

        lax.fori_loop(0, (need + _L - 1) // _L, rbody, 0)

    @pl.when(jnp.logical_not(cond))
    def _():
        @plsc.parallel_loop(0, _N // (_L * _U), unroll=2)
        def _(i):
            for u in range(_U):
                sl = pl.ds((i * _U + u) * _L, _L)
                row_v[sl] = jnp.clip(row_v[sl], 0.0, 1.0)

    pltpu.sync_copy(row_v, out_hbm.at[row])


def _sc_body(scores_hbm, kvec_hbm, out_hbm, row_v, cand_v, tie_v, hist_v, kv_v):
    wid = lax.axis_index("s") * 2 + lax.axis_index("c")
    pltpu.sync_copy(kvec_hbm, kv_v)
    k_s = jnp.sum(kv_v[...]) * 0.0625
    for r in range(2):
        _process_row(wid * 2 + r, row_v, cand_v, tie_v, hist_v, k_s,
                     scores_hbm, out_hbm)


def kernel(scores, k):
    kvec = jnp.broadcast_to(jnp.asarray(k, jnp.float32), (_L,))
    mesh = plsc.VectorSubcoreMesh(core_axis_name="c", subcore_axis_name="s")
    fn = functools.partial(
        pl.kernel,
        mesh=mesh,
        out_type=jax.ShapeDtypeStruct(scores.shape, scores.dtype),
        scratch_types=[
            pltpu.VMEM((_N,), jnp.float32),
            pltpu.VMEM((_N,), jnp.float32),
            pltpu.VMEM((_N,), jnp.int32),
            pltpu.VMEM((_NB * _L,), jnp.int32),
            pltpu.VMEM((_L,), jnp.float32),
        ],
        compiler_params=pltpu.CompilerParams(needs_layout_passes=False),
    )(_sc_body)
    return fn(scores, kvec)
